# Initial kernel scaffold; baseline (speedup 1.0000x reference)
import jax
import jax.numpy as jnp
from jax.experimental import pallas as pl


def kernel(x_num, x_cat, W, b, tables):
    raise NotImplementedError("write your pallas kernel here")


# speedup vs baseline: 1.3287x; 1.3287x over previous
import functools

import jax
import jax.numpy as jnp
from jax import lax
from jax.experimental import pallas as pl
from jax.experimental.pallas import tpu as pltpu
from jax.experimental.pallas import tpu_sc as plsc

B = 16384
N_NUM = 128
N_CAT = 26
VOCAB = 100000

NW = 32
BPW = B // NW
LANES = 16
NVEC = BPW // LANES
CHUNK = 128
NCH = BPW // CHUNK


def _sc_embed_sum(xcat_t, tflat):
    mesh = plsc.VectorSubcoreMesh(core_axis_name="c", subcore_axis_name="s")

    @functools.partial(
        pl.kernel,
        mesh=mesh,
        out_type=jax.ShapeDtypeStruct((B,), jnp.float32),
        scratch_types=[
            pltpu.VMEM((N_CAT, BPW), jnp.int32),
            pltpu.VMEM((N_CAT, BPW), jnp.float32),
            pltpu.VMEM((BPW,), jnp.float32),
            pltpu.SemaphoreType.DMA,
        ],
    )
    def k(xcat_hbm, tflat_hbm, out_hbm, idx_v, g_v, acc_v, sem):
        wid = lax.axis_index("s") * 2 + lax.axis_index("c")
        base = wid * BPW
        pltpu.sync_copy(xcat_hbm.at[:, pl.ds(base, BPW)], idx_v)

        for i in range(1, N_CAT):
            def obody(j, _, i=i):
                sl = pl.ds(j * LANES, LANES)
                idx_v[i, sl] = idx_v[i, sl] + (i * VOCAB)
                return 0
            lax.fori_loop(0, NVEC, obody, 0)

        copies = []
        for i in range(N_CAT):
            for c in range(NCH):
                sl = pl.ds(c * CHUNK, CHUNK)
                copies.append(
                    pltpu.async_copy(
                        tflat_hbm.at[idx_v.at[i, sl]], g_v.at[i, sl], sem
                    )
                )
        for cp in copies:
            cp.wait()

        def rbody(j, _):
            sl = pl.ds(j * LANES, LANES)
            acc = g_v[0, sl]
            for i in range(1, N_CAT):
                acc = acc + g_v[i, sl]
            acc_v[sl] = acc
            return 0
        lax.fori_loop(0, NVEC, rbody, 0)

        pltpu.sync_copy(acc_v, out_hbm.at[pl.ds(base, BPW)])

    return k(xcat_t, tflat)


def _tc_matvec(x, w):
    blk = 2048

    def body(x_ref, w_ref, o_ref):
        o_ref[...] = jnp.dot(
            x_ref[...], w_ref[...], preferred_element_type=jnp.float32
        )

    return pl.pallas_call(
        body,
        grid=(B // blk,),
        in_specs=[
            pl.BlockSpec((blk, N_NUM), lambda i: (i, 0)),
            pl.BlockSpec((N_NUM, 1), lambda i: (0, 0)),
        ],
        out_specs=pl.BlockSpec((blk, 1), lambda i: (i, 0)),
        out_shape=jax.ShapeDtypeStruct((B, 1), jnp.float32),
    )(x, w)


def kernel(x_num, x_cat, W, b, tables):
    xcat_t = x_cat.T.astype(jnp.int32)
    tflat = tables.reshape(N_CAT * VOCAB)
    emb = _sc_embed_sum(xcat_t, tflat)
    lin = _tc_matvec(x_num, W)
    return lin + emb[:, None] + b
